# baseline (speedup 1.0000x reference)
import functools

import jax
import jax.numpy as jnp
from jax import lax
from jax.experimental import pallas as pl
from jax.experimental.pallas import tpu as pltpu
from jax.experimental.pallas import tpu_sc as plsc

VOCAB = 100000
EMB_DIM = 128
HIDDEN = 512
OUT_DIM = 128
BATCH = 4096
SEQ = 200

NTOK = BATCH * SEQ
NC, NS = 2, 16
NW = NC * NS
IDX_ROWS = NTOK // 128

NCHUNK = 5
CH_TOK = NTOK // NCHUNK
CH_ROWS = IDX_ROWS // NCHUNK
ROWS_PER_W = CH_ROWS // NW
K = 8
HALF = K // 2
GROUPS = ROWS_PER_W // K

TILE = 4096
CH_STEPS = CH_TOK // TILE


@functools.partial(
    pl.kernel,
    mesh=plsc.VectorSubcoreMesh(core_axis_name="c", subcore_axis_name="s"),
    out_type=jax.ShapeDtypeStruct((CH_TOK, EMB_DIM), jnp.float32),
    scratch_types=[
        pltpu.VMEM((K, 128), jnp.int32),
        pltpu.VMEM((HALF * 128, EMB_DIM), jnp.float32),
        pltpu.SemaphoreType.DMA,
    ],
)
def _sc_gather(x2_hbm, emb_hbm, out_hbm, idx_v, rows_v, sem):
    wid = lax.axis_index("s") * NC + lax.axis_index("c")
    row0 = wid * ROWS_PER_W

    def group(g, carry):
        r = row0 + g * K
        pltpu.sync_copy(x2_hbm.at[pl.ds(r, K)], idx_v)
        for w in range(2):
            for j in range(HALF):
                pltpu.async_copy(
                    emb_hbm.at[idx_v.at[w * HALF + j]],
                    rows_v.at[pl.ds(j * 128, 128)], sem,
                )
            for j in range(HALF):
                pltpu.make_async_copy(
                    emb_hbm.at[idx_v.at[w * HALF + j]],
                    rows_v.at[pl.ds(j * 128, 128)], sem,
                ).wait()
            pltpu.sync_copy(
                rows_v, out_hbm.at[pl.ds((r + w * HALF) * 128, HALF * 128)]
            )
        return carry

    lax.fori_loop(0, GROUPS, group, 0)


def _mlp_body(*refs):
    if len(refs) == 9:
        refs = refs[1:]
    e_ref, w1_ref, b1_ref, w2_ref, b2_ref, w3_ref, b3_ref, o_ref = refs
    h = jnp.dot(e_ref[...].astype(jnp.bfloat16), w1_ref[...],
                preferred_element_type=jnp.float32).astype(jnp.bfloat16)
    h = jnp.maximum(h + b1_ref[...], jnp.bfloat16(0.0))
    h = jnp.dot(h, w2_ref[...],
                preferred_element_type=jnp.float32).astype(jnp.bfloat16)
    h = jnp.maximum(h + b2_ref[...], jnp.bfloat16(0.0))
    o = jnp.dot(h, w3_ref[...], preferred_element_type=jnp.float32)
    o_ref[...] = o + b3_ref[...]


def _mlp_chunk_call(chunk):
    step0 = chunk * CH_STEPS
    alias = ([pl.BlockSpec(memory_space=pl.ANY)] if chunk else [])
    return pl.pallas_call(
        _mlp_body,
        grid=(CH_STEPS,),
        in_specs=alias + [
            pl.BlockSpec((TILE, EMB_DIM), lambda i: (i, 0)),
            pl.BlockSpec((EMB_DIM, HIDDEN), lambda i: (0, 0)),
            pl.BlockSpec((1, HIDDEN), lambda i: (0, 0)),
            pl.BlockSpec((HIDDEN, HIDDEN), lambda i: (0, 0)),
            pl.BlockSpec((1, HIDDEN), lambda i: (0, 0)),
            pl.BlockSpec((HIDDEN, OUT_DIM), lambda i: (0, 0)),
            pl.BlockSpec((1, OUT_DIM), lambda i: (0, 0)),
        ],
        out_specs=pl.BlockSpec((TILE, OUT_DIM), lambda i: (step0 + i, 0)),
        out_shape=jax.ShapeDtypeStruct((NTOK, OUT_DIM), jnp.float32),
        input_output_aliases={0: 0} if chunk else {},
        compiler_params=pltpu.CompilerParams(
            dimension_semantics=("arbitrary",)
        ),
    )


def kernel(x, emb, W1, b1, W2, b2, W3, b3):
    x2 = x.reshape(IDX_ROWS, 128).astype(jnp.int32)
    w_args = (
        W1.astype(jnp.bfloat16), b1.astype(jnp.bfloat16).reshape(1, HIDDEN),
        W2.astype(jnp.bfloat16), b2.astype(jnp.bfloat16).reshape(1, HIDDEN),
        W3.astype(jnp.bfloat16), b3.reshape(1, OUT_DIM),
    )
    es = [
        _sc_gather(lax.slice_in_dim(x2, c * CH_ROWS, (c + 1) * CH_ROWS), emb)
        for c in range(NCHUNK)
    ]
    out = _mlp_chunk_call(0)(es[0], *w_args)
    for c in range(1, NCHUNK):
        out = _mlp_chunk_call(c)(out, es[c], *w_args)
    return out.reshape(BATCH, SEQ, OUT_DIM)


# speedup vs baseline: 1.0262x; 1.0262x over previous
import functools

import jax
import jax.numpy as jnp
from jax import lax
from jax.experimental import pallas as pl
from jax.experimental.pallas import tpu as pltpu
from jax.experimental.pallas import tpu_sc as plsc

VOCAB = 100000
EMB_DIM = 128
HIDDEN = 512
OUT_DIM = 128
BATCH = 4096
SEQ = 200

NTOK = BATCH * SEQ
NC, NS = 2, 16
NW = NC * NS
IDX_ROWS = NTOK // 128

CHUNK_ROWS = (256, 1536, 1536, 1536, 1536)
NCHUNK = len(CHUNK_ROWS)
K = 8
HALF = K // 2

TILE = 8192


@functools.cache
def _sc_gather(ch_rows):
    rows_per_w = ch_rows // NW
    groups = rows_per_w // K

    @functools.partial(
        pl.kernel,
        mesh=plsc.VectorSubcoreMesh(core_axis_name="c", subcore_axis_name="s"),
        out_type=jax.ShapeDtypeStruct((ch_rows * 128, EMB_DIM), jnp.float32),
        scratch_types=[
            pltpu.VMEM((K, 128), jnp.int32),
            pltpu.VMEM((HALF * 128, EMB_DIM), jnp.float32),
            pltpu.SemaphoreType.DMA,
        ],
    )
    def gather(x2_hbm, emb_hbm, out_hbm, idx_v, rows_v, sem):
        wid = lax.axis_index("s") * NC + lax.axis_index("c")
        row0 = wid * rows_per_w

        def group(g, carry):
            r = row0 + g * K
            pltpu.sync_copy(x2_hbm.at[pl.ds(r, K)], idx_v)
            for w in range(2):
                for j in range(HALF):
                    pltpu.async_copy(
                        emb_hbm.at[idx_v.at[w * HALF + j]],
                        rows_v.at[pl.ds(j * 128, 128)], sem,
                    )
                for j in range(HALF):
                    pltpu.make_async_copy(
                        emb_hbm.at[idx_v.at[w * HALF + j]],
                        rows_v.at[pl.ds(j * 128, 128)], sem,
                    ).wait()
                pltpu.sync_copy(
                    rows_v, out_hbm.at[pl.ds((r + w * HALF) * 128, HALF * 128)]
                )
            return carry

        lax.fori_loop(0, groups, group, 0)

    return gather


def _mlp_body(*refs):
    if len(refs) == 9:
        refs = refs[1:]
    e_ref, w1_ref, b1_ref, w2_ref, b2_ref, w3_ref, b3_ref, o_ref = refs
    h = jnp.dot(e_ref[...].astype(jnp.bfloat16), w1_ref[...],
                preferred_element_type=jnp.float32).astype(jnp.bfloat16)
    h = jnp.maximum(h + b1_ref[...], jnp.bfloat16(0.0))
    h = jnp.dot(h, w2_ref[...],
                preferred_element_type=jnp.float32).astype(jnp.bfloat16)
    h = jnp.maximum(h + b2_ref[...], jnp.bfloat16(0.0))
    o = jnp.dot(h, w3_ref[...], preferred_element_type=jnp.float32)
    o_ref[...] = o + b3_ref[...]


def _mlp_chunk_call(chunk, step0, steps):
    alias = ([pl.BlockSpec(memory_space=pl.ANY)] if chunk else [])
    return pl.pallas_call(
        _mlp_body,
        grid=(steps,),
        in_specs=alias + [
            pl.BlockSpec((TILE, EMB_DIM), lambda i: (i, 0)),
            pl.BlockSpec((EMB_DIM, HIDDEN), lambda i: (0, 0)),
            pl.BlockSpec((1, HIDDEN), lambda i: (0, 0)),
            pl.BlockSpec((HIDDEN, HIDDEN), lambda i: (0, 0)),
            pl.BlockSpec((1, HIDDEN), lambda i: (0, 0)),
            pl.BlockSpec((HIDDEN, OUT_DIM), lambda i: (0, 0)),
            pl.BlockSpec((1, OUT_DIM), lambda i: (0, 0)),
        ],
        out_specs=pl.BlockSpec((TILE, OUT_DIM), lambda i: (step0 + i, 0)),
        out_shape=jax.ShapeDtypeStruct((NTOK, OUT_DIM), jnp.float32),
        input_output_aliases={0: 0} if chunk else {},
        compiler_params=pltpu.CompilerParams(
            dimension_semantics=("arbitrary",)
        ),
    )


def kernel(x, emb, W1, b1, W2, b2, W3, b3):
    x2 = x.reshape(IDX_ROWS, 128).astype(jnp.int32)
    w_args = (
        W1.astype(jnp.bfloat16), b1.astype(jnp.bfloat16).reshape(1, HIDDEN),
        W2.astype(jnp.bfloat16), b2.astype(jnp.bfloat16).reshape(1, HIDDEN),
        W3.astype(jnp.bfloat16), b3.reshape(1, OUT_DIM),
    )
    es, row0 = [], 0
    for rows in CHUNK_ROWS:
        es.append(_sc_gather(rows)(
            lax.slice_in_dim(x2, row0, row0 + rows), emb))
        row0 += rows
    out, step0 = None, 0
    for c, rows in enumerate(CHUNK_ROWS):
        steps = rows * 128 // TILE
        call = _mlp_chunk_call(c, step0, steps)
        out = call(es[c], *w_args) if c == 0 else call(out, es[c], *w_args)
        step0 += steps
    return out.reshape(BATCH, SEQ, OUT_DIM)


# speedup vs baseline: 1.0516x; 1.0248x over previous
import functools

import jax
import jax.numpy as jnp
from jax import lax
from jax.experimental import pallas as pl
from jax.experimental.pallas import tpu as pltpu
from jax.experimental.pallas import tpu_sc as plsc

VOCAB = 100000
EMB_DIM = 128
HIDDEN = 512
OUT_DIM = 128
BATCH = 4096
SEQ = 200

NTOK = BATCH * SEQ
NC, NS = 2, 16
NW = NC * NS
IDX_ROWS = NTOK // 128

CHUNK_ROWS = (256, 256, 512, 1024, 1536, 1536, 1280)
NCHUNK = len(CHUNK_ROWS)
K = 8
HALF = K // 2

TILE = 8192


@functools.cache
def _sc_gather(ch_rows):
    rows_per_w = ch_rows // NW
    groups = rows_per_w // K

    @functools.partial(
        pl.kernel,
        mesh=plsc.VectorSubcoreMesh(core_axis_name="c", subcore_axis_name="s"),
        out_type=jax.ShapeDtypeStruct((ch_rows * 128, EMB_DIM), jnp.float32),
        scratch_types=[
            pltpu.VMEM((K, 128), jnp.int32),
            pltpu.VMEM((HALF * 128, EMB_DIM), jnp.float32),
            pltpu.SemaphoreType.DMA,
        ],
    )
    def gather(x2_hbm, emb_hbm, out_hbm, idx_v, rows_v, sem):
        wid = lax.axis_index("s") * NC + lax.axis_index("c")
        row0 = wid * rows_per_w

        def group(g, carry):
            r = row0 + g * K
            pltpu.sync_copy(x2_hbm.at[pl.ds(r, K)], idx_v)
            for w in range(2):
                for j in range(HALF):
                    pltpu.async_copy(
                        emb_hbm.at[idx_v.at[w * HALF + j]],
                        rows_v.at[pl.ds(j * 128, 128)], sem,
                    )
                for j in range(HALF):
                    pltpu.make_async_copy(
                        emb_hbm.at[idx_v.at[w * HALF + j]],
                        rows_v.at[pl.ds(j * 128, 128)], sem,
                    ).wait()
                pltpu.sync_copy(
                    rows_v, out_hbm.at[pl.ds((r + w * HALF) * 128, HALF * 128)]
                )
            return carry

        lax.fori_loop(0, groups, group, 0)

    return gather


def _mlp_body(*refs):
    if len(refs) == 9:
        refs = refs[1:]
    e_ref, w1_ref, b1_ref, w2_ref, b2_ref, w3_ref, b3_ref, o_ref = refs
    h = jnp.dot(e_ref[...].astype(jnp.bfloat16), w1_ref[...],
                preferred_element_type=jnp.float32).astype(jnp.bfloat16)
    h = jnp.maximum(h + b1_ref[...], jnp.bfloat16(0.0))
    h = jnp.dot(h, w2_ref[...],
                preferred_element_type=jnp.float32).astype(jnp.bfloat16)
    h = jnp.maximum(h + b2_ref[...], jnp.bfloat16(0.0))
    o = jnp.dot(h, w3_ref[...], preferred_element_type=jnp.float32)
    o_ref[...] = o + b3_ref[...]


def _mlp_chunk_call(chunk, step0, steps):
    alias = ([pl.BlockSpec(memory_space=pl.ANY)] if chunk else [])
    return pl.pallas_call(
        _mlp_body,
        grid=(steps,),
        in_specs=alias + [
            pl.BlockSpec((TILE, EMB_DIM), lambda i: (i, 0)),
            pl.BlockSpec((EMB_DIM, HIDDEN), lambda i: (0, 0)),
            pl.BlockSpec((1, HIDDEN), lambda i: (0, 0)),
            pl.BlockSpec((HIDDEN, HIDDEN), lambda i: (0, 0)),
            pl.BlockSpec((1, HIDDEN), lambda i: (0, 0)),
            pl.BlockSpec((HIDDEN, OUT_DIM), lambda i: (0, 0)),
            pl.BlockSpec((1, OUT_DIM), lambda i: (0, 0)),
        ],
        out_specs=pl.BlockSpec((TILE, OUT_DIM), lambda i: (step0 + i, 0)),
        out_shape=jax.ShapeDtypeStruct((NTOK, OUT_DIM), jnp.float32),
        input_output_aliases={0: 0} if chunk else {},
        compiler_params=pltpu.CompilerParams(
            dimension_semantics=("arbitrary",)
        ),
    )


def kernel(x, emb, W1, b1, W2, b2, W3, b3):
    x2 = x.reshape(IDX_ROWS, 128).astype(jnp.int32)
    w_args = (
        W1.astype(jnp.bfloat16), b1.astype(jnp.bfloat16).reshape(1, HIDDEN),
        W2.astype(jnp.bfloat16), b2.astype(jnp.bfloat16).reshape(1, HIDDEN),
        W3.astype(jnp.bfloat16), b3.reshape(1, OUT_DIM),
    )
    es, row0 = [], 0
    for rows in CHUNK_ROWS:
        es.append(_sc_gather(rows)(
            lax.slice_in_dim(x2, row0, row0 + rows), emb))
        row0 += rows
    out, step0 = None, 0
    for c, rows in enumerate(CHUNK_ROWS):
        steps = rows * 128 // TILE
        call = _mlp_chunk_call(c, step0, steps)
        out = call(es[c], *w_args) if c == 0 else call(out, es[c], *w_args)
        step0 += steps
    return out.reshape(BATCH, SEQ, OUT_DIM)


# speedup vs baseline: 1.0518x; 1.0002x over previous
import functools

import jax
import jax.numpy as jnp
from jax import lax
from jax.experimental import pallas as pl
from jax.experimental.pallas import tpu as pltpu
from jax.experimental.pallas import tpu_sc as plsc

VOCAB = 100000
EMB_DIM = 128
HIDDEN = 512
OUT_DIM = 128
BATCH = 4096
SEQ = 200

NTOK = BATCH * SEQ
NC, NS = 2, 16
NW = NC * NS
IDX_ROWS = NTOK // 128

CHUNK_ROWS = (256, 256, 512, 1024, 1536, 1536, 1280)
NCHUNK = len(CHUNK_ROWS)
K = 8
HALF = K // 2

TILE = 8192


@functools.cache
def _sc_gather(ch_rows):
    rows_per_w = ch_rows // NW
    groups = rows_per_w // K

    @functools.partial(
        pl.kernel,
        mesh=plsc.VectorSubcoreMesh(core_axis_name="c", subcore_axis_name="s"),
        out_type=jax.ShapeDtypeStruct((ch_rows * 128, EMB_DIM), jnp.float32),
        scratch_types=[
            pltpu.VMEM((K, 128), jnp.int32),
            pltpu.VMEM((HALF * 128, EMB_DIM), jnp.float32),
            pltpu.SemaphoreType.DMA,
        ],
    )
    def gather(x2_hbm, emb_hbm, out_hbm, idx_v, rows_v, sem):
        wid = lax.axis_index("s") * NC + lax.axis_index("c")
        row0 = wid * rows_per_w

        def group(g, carry):
            r = row0 + g * K
            pltpu.sync_copy(x2_hbm.at[pl.ds(r, K)], idx_v)
            for w in range(2):
                for j in range(HALF):
                    pltpu.async_copy(
                        emb_hbm.at[idx_v.at[w * HALF + j]],
                        rows_v.at[pl.ds(j * 128, 128)], sem,
                    )
                for j in range(HALF):
                    pltpu.make_async_copy(
                        emb_hbm.at[idx_v.at[w * HALF + j]],
                        rows_v.at[pl.ds(j * 128, 128)], sem,
                    ).wait()
                pltpu.sync_copy(
                    rows_v, out_hbm.at[pl.ds((r + w * HALF) * 128, HALF * 128)]
                )
            return carry

        lax.fori_loop(0, groups, group, 0)

    return gather


def _mlp_body(*refs):
    if len(refs) == 9:
        refs = refs[1:]
    e_ref, w1_ref, b1_ref, w2_ref, b2_ref, w3_ref, b3_ref, o_ref = refs
    h = jnp.dot(e_ref[...].astype(jnp.bfloat16), w1_ref[...],
                preferred_element_type=jnp.float32).astype(jnp.bfloat16)
    h = jnp.maximum(h + b1_ref[...], jnp.bfloat16(0.0))
    h = jnp.dot(h, w2_ref[...],
                preferred_element_type=jnp.float32).astype(jnp.bfloat16)
    h = jnp.maximum(h + b2_ref[...], jnp.bfloat16(0.0))
    o = jnp.dot(h, w3_ref[...], preferred_element_type=jnp.float32)
    o_ref[...] = o + b3_ref[...]


def _mlp_chunk_call(chunk, step0, steps):
    alias = ([pl.BlockSpec(memory_space=pl.ANY)] if chunk else [])
    return pl.pallas_call(
        _mlp_body,
        grid=(steps,),
        in_specs=alias + [
            pl.BlockSpec((TILE, EMB_DIM), lambda i: (i, 0)),
            pl.BlockSpec((EMB_DIM, HIDDEN), lambda i: (0, 0)),
            pl.BlockSpec((1, HIDDEN), lambda i: (0, 0)),
            pl.BlockSpec((HIDDEN, HIDDEN), lambda i: (0, 0)),
            pl.BlockSpec((1, HIDDEN), lambda i: (0, 0)),
            pl.BlockSpec((HIDDEN, OUT_DIM), lambda i: (0, 0)),
            pl.BlockSpec((1, OUT_DIM), lambda i: (0, 0)),
        ],
        out_specs=pl.BlockSpec((TILE, OUT_DIM), lambda i: (step0 + i, 0)),
        out_shape=jax.ShapeDtypeStruct((NTOK, OUT_DIM), jnp.float32),
        input_output_aliases={0: 0} if chunk else {},
        compiler_params=pltpu.CompilerParams(
            dimension_semantics=("parallel",)
        ),
    )


def kernel(x, emb, W1, b1, W2, b2, W3, b3):
    x2 = x.reshape(IDX_ROWS, 128).astype(jnp.int32)
    w_args = (
        W1.astype(jnp.bfloat16), b1.astype(jnp.bfloat16).reshape(1, HIDDEN),
        W2.astype(jnp.bfloat16), b2.astype(jnp.bfloat16).reshape(1, HIDDEN),
        W3.astype(jnp.bfloat16), b3.reshape(1, OUT_DIM),
    )
    es, row0 = [], 0
    for rows in CHUNK_ROWS:
        es.append(_sc_gather(rows)(
            lax.slice_in_dim(x2, row0, row0 + rows), emb))
        row0 += rows
    out, step0 = None, 0
    for c, rows in enumerate(CHUNK_ROWS):
        steps = rows * 128 // TILE
        call = _mlp_chunk_call(c, step0, steps)
        out = call(es[c], *w_args) if c == 0 else call(out, es[c], *w_args)
        step0 += steps
    return out.reshape(BATCH, SEQ, OUT_DIM)


# speedup vs baseline: 2.1800x; 2.0727x over previous
import functools

import jax
import jax.numpy as jnp
from jax import lax
from jax.experimental import pallas as pl
from jax.experimental.pallas import tpu as pltpu
from jax.experimental.pallas import tpu_sc as plsc

VOCAB = 100000
EMB_DIM = 128
HIDDEN = 512
OUT_DIM = 128
BATCH = 4096
SEQ = 200

NTOK = BATCH * SEQ
NC, NS = 2, 16
NW = NC * NS
IDX_ROWS = NTOK // 128
ROWS_PER_W = IDX_ROWS // NW
K = 8
HALF = K // 2
GROUPS = ROWS_PER_W // K

TILE = 10000


@functools.partial(
    pl.kernel,
    mesh=plsc.VectorSubcoreMesh(core_axis_name="c", subcore_axis_name="s"),
    out_type=jax.ShapeDtypeStruct((NTOK, OUT_DIM), jnp.float32),
    scratch_types=[
        pltpu.VMEM((K, 128), jnp.int32),
        pltpu.VMEM((HALF * 128, OUT_DIM), jnp.float32),
        pltpu.SemaphoreType.DMA,
    ],
)
def _sc_gather(x2_hbm, tab_hbm, out_hbm, idx_v, rows_v, sem):
    wid = lax.axis_index("s") * NC + lax.axis_index("c")
    row0 = wid * ROWS_PER_W

    def group(g, carry):
        r = row0 + g * K
        pltpu.sync_copy(x2_hbm.at[pl.ds(r, K)], idx_v)
        for w in range(2):
            for j in range(HALF):
                pltpu.async_copy(
                    tab_hbm.at[idx_v.at[w * HALF + j]],
                    rows_v.at[pl.ds(j * 128, 128)], sem,
                )
            for j in range(HALF):
                pltpu.make_async_copy(
                    tab_hbm.at[idx_v.at[w * HALF + j]],
                    rows_v.at[pl.ds(j * 128, 128)], sem,
                ).wait()
            pltpu.sync_copy(
                rows_v, out_hbm.at[pl.ds((r + w * HALF) * 128, HALF * 128)]
            )
        return carry

    lax.fori_loop(0, GROUPS, group, 0)


def _mlp_body(e_ref, w1_ref, b1_ref, w2_ref, b2_ref, w3_ref, b3_ref, o_ref):
    h = jnp.dot(e_ref[...].astype(jnp.bfloat16), w1_ref[...],
                preferred_element_type=jnp.float32).astype(jnp.bfloat16)
    h = jnp.maximum(h + b1_ref[...], jnp.bfloat16(0.0))
    h = jnp.dot(h, w2_ref[...],
                preferred_element_type=jnp.float32).astype(jnp.bfloat16)
    h = jnp.maximum(h + b2_ref[...], jnp.bfloat16(0.0))
    o = jnp.dot(h, w3_ref[...], preferred_element_type=jnp.float32)
    o_ref[...] = o + b3_ref[...]


def _mlp_table(emb, W1, b1, W2, b2, W3, b3):
    return pl.pallas_call(
        _mlp_body,
        grid=(VOCAB // TILE,),
        in_specs=[
            pl.BlockSpec((TILE, EMB_DIM), lambda i: (i, 0)),
            pl.BlockSpec((EMB_DIM, HIDDEN), lambda i: (0, 0)),
            pl.BlockSpec((1, HIDDEN), lambda i: (0, 0)),
            pl.BlockSpec((HIDDEN, HIDDEN), lambda i: (0, 0)),
            pl.BlockSpec((1, HIDDEN), lambda i: (0, 0)),
            pl.BlockSpec((HIDDEN, OUT_DIM), lambda i: (0, 0)),
            pl.BlockSpec((1, OUT_DIM), lambda i: (0, 0)),
        ],
        out_specs=pl.BlockSpec((TILE, OUT_DIM), lambda i: (i, 0)),
        out_shape=jax.ShapeDtypeStruct((VOCAB, OUT_DIM), jnp.float32),
        compiler_params=pltpu.CompilerParams(
            dimension_semantics=("parallel",)
        ),
    )(emb, W1.astype(jnp.bfloat16), b1.astype(jnp.bfloat16).reshape(1, HIDDEN),
      W2.astype(jnp.bfloat16), b2.astype(jnp.bfloat16).reshape(1, HIDDEN),
      W3.astype(jnp.bfloat16), b3.reshape(1, OUT_DIM))


def kernel(x, emb, W1, b1, W2, b2, W3, b3):
    x2 = x.reshape(IDX_ROWS, 128).astype(jnp.int32)
    tab = _mlp_table(emb, W1, b1, W2, b2, W3, b3)
    out = _sc_gather(x2, tab)
    return out.reshape(BATCH, SEQ, OUT_DIM)


# speedup vs baseline: 2.3248x; 1.0664x over previous
import functools

import jax
import jax.numpy as jnp
from jax import lax
from jax.experimental import pallas as pl
from jax.experimental.pallas import tpu as pltpu
from jax.experimental.pallas import tpu_sc as plsc

VOCAB = 100000
EMB_DIM = 128
HIDDEN = 512
OUT_DIM = 128
BATCH = 4096
SEQ = 200

NTOK = BATCH * SEQ
NC, NS = 2, 16
NW = NC * NS
IDX_ROWS = NTOK // 128
ROWS_PER_W = IDX_ROWS // NW
Q = 2
PAIRS = ROWS_PER_W // (2 * Q)

TILE = 10000


@functools.partial(
    pl.kernel,
    mesh=plsc.VectorSubcoreMesh(core_axis_name="c", subcore_axis_name="s"),
    out_type=jax.ShapeDtypeStruct((NTOK, OUT_DIM), jnp.float32),
    scratch_types=[
        pltpu.VMEM((ROWS_PER_W, 128), jnp.int32),
        pltpu.VMEM((Q * 128, OUT_DIM), jnp.float32),
        pltpu.VMEM((Q * 128, OUT_DIM), jnp.float32),
        pltpu.SemaphoreType.DMA,
        pltpu.SemaphoreType.DMA,
        pltpu.SemaphoreType.DMA,
        pltpu.SemaphoreType.DMA,
    ],
)
def _sc_gather(x2_hbm, tab_hbm, out_hbm, idx_v, buf0, buf1, g0, g1, s0, s1):
    wid = lax.axis_index("s") * NC + lax.axis_index("c")
    row0 = wid * ROWS_PER_W
    pltpu.sync_copy(x2_hbm.at[pl.ds(row0, ROWS_PER_W)], idx_v)
    bufs, gsems, ssems = (buf0, buf1), (g0, g1), (s0, s1)

    def gather_args(q, b):
        return [
            (tab_hbm.at[idx_v.at[q * Q + j]],
             bufs[b].at[pl.ds(j * 128, 128)], gsems[b])
            for j in range(Q)
        ]

    def store_args(q, b):
        return (bufs[b],
                out_hbm.at[pl.ds((row0 + q * Q) * 128, Q * 128)], ssems[b])

    def pair(p, carry):
        qa, qb = 2 * p, 2 * p + 1
        for b, q in ((0, qa), (1, qb)):
            @pl.when(p > 0)
            def _():
                pltpu.make_async_copy(*store_args(q, b)).wait()
            for a in gather_args(q, b):
                pltpu.async_copy(*a)
        for b, q in ((0, qa), (1, qb)):
            for a in gather_args(q, b):
                pltpu.make_async_copy(*a).wait()
            pltpu.async_copy(*store_args(q, b))
        return carry

    lax.fori_loop(0, PAIRS, pair, 0)
    pltpu.make_async_copy(*store_args(0, 0)).wait()
    pltpu.make_async_copy(*store_args(0, 1)).wait()


def _mlp_body(e_ref, w1_ref, b1_ref, w2_ref, b2_ref, w3_ref, b3_ref, o_ref):
    h = jnp.dot(e_ref[...].astype(jnp.bfloat16), w1_ref[...],
                preferred_element_type=jnp.float32).astype(jnp.bfloat16)
    h = jnp.maximum(h + b1_ref[...], jnp.bfloat16(0.0))
    h = jnp.dot(h, w2_ref[...],
                preferred_element_type=jnp.float32).astype(jnp.bfloat16)
    h = jnp.maximum(h + b2_ref[...], jnp.bfloat16(0.0))
    o = jnp.dot(h, w3_ref[...], preferred_element_type=jnp.float32)
    o_ref[...] = o + b3_ref[...]


def _mlp_table(emb, W1, b1, W2, b2, W3, b3):
    return pl.pallas_call(
        _mlp_body,
        grid=(VOCAB // TILE,),
        in_specs=[
            pl.BlockSpec((TILE, EMB_DIM), lambda i: (i, 0)),
            pl.BlockSpec((EMB_DIM, HIDDEN), lambda i: (0, 0)),
            pl.BlockSpec((1, HIDDEN), lambda i: (0, 0)),
            pl.BlockSpec((HIDDEN, HIDDEN), lambda i: (0, 0)),
            pl.BlockSpec((1, HIDDEN), lambda i: (0, 0)),
            pl.BlockSpec((HIDDEN, OUT_DIM), lambda i: (0, 0)),
            pl.BlockSpec((1, OUT_DIM), lambda i: (0, 0)),
        ],
        out_specs=pl.BlockSpec((TILE, OUT_DIM), lambda i: (i, 0)),
        out_shape=jax.ShapeDtypeStruct((VOCAB, OUT_DIM), jnp.float32),
        compiler_params=pltpu.CompilerParams(
            dimension_semantics=("parallel",)
        ),
    )(emb, W1.astype(jnp.bfloat16), b1.astype(jnp.bfloat16).reshape(1, HIDDEN),
      W2.astype(jnp.bfloat16), b2.astype(jnp.bfloat16).reshape(1, HIDDEN),
      W3.astype(jnp.bfloat16), b3.reshape(1, OUT_DIM))


def kernel(x, emb, W1, b1, W2, b2, W3, b3):
    x2 = x.reshape(IDX_ROWS, 128).astype(jnp.int32)
    tab = _mlp_table(emb, W1, b1, W2, b2, W3, b3)
    out = _sc_gather(x2, tab)
    return out.reshape(BATCH, SEQ, OUT_DIM)


# speedup vs baseline: 2.3586x; 1.0145x over previous
import functools

import jax
import jax.numpy as jnp
from jax import lax
from jax.experimental import pallas as pl
from jax.experimental.pallas import tpu as pltpu
from jax.experimental.pallas import tpu_sc as plsc

VOCAB = 100000
EMB_DIM = 128
HIDDEN = 512
OUT_DIM = 128
BATCH = 4096
SEQ = 200

NTOK = BATCH * SEQ
NC, NS = 2, 16
NW = NC * NS
IDX_ROWS = NTOK // 128
ROWS_PER_W = IDX_ROWS // NW
Q = 2
NBUF = 3
WAVES = ROWS_PER_W // Q
RING_ITERS = WAVES // NBUF

TILE = 10000


@functools.partial(
    pl.kernel,
    mesh=plsc.VectorSubcoreMesh(core_axis_name="c", subcore_axis_name="s"),
    out_type=jax.ShapeDtypeStruct((NTOK, OUT_DIM), jnp.float32),
    scratch_types=[
        pltpu.VMEM((ROWS_PER_W, 128), jnp.int32),
        pltpu.VMEM((Q * 128, OUT_DIM), jnp.float32),
        pltpu.VMEM((Q * 128, OUT_DIM), jnp.float32),
        pltpu.VMEM((Q * 128, OUT_DIM), jnp.float32),
        pltpu.SemaphoreType.DMA,
        pltpu.SemaphoreType.DMA,
        pltpu.SemaphoreType.DMA,
        pltpu.SemaphoreType.DMA,
        pltpu.SemaphoreType.DMA,
        pltpu.SemaphoreType.DMA,
    ],
)
def _sc_gather(x2_hbm, tab_hbm, out_hbm, idx_v, buf0, buf1, buf2,
               g0, g1, g2, s0, s1, s2):
    wid = lax.axis_index("s") * NC + lax.axis_index("c")
    row0 = wid * ROWS_PER_W
    pltpu.sync_copy(x2_hbm.at[pl.ds(row0, ROWS_PER_W)], idx_v)
    bufs, gsems, ssems = (buf0, buf1, buf2), (g0, g1, g2), (s0, s1, s2)

    def gather_args(q, b):
        return [
            (tab_hbm.at[idx_v.at[q * Q + j]],
             bufs[b].at[pl.ds(j * 128, 128)], gsems[b])
            for j in range(Q)
        ]

    def store_args(q, b):
        return (bufs[b],
                out_hbm.at[pl.ds((row0 + q * Q) * 128, Q * 128)], ssems[b])

    def turn(p, carry):
        for b in range(NBUF):
            q = NBUF * p + b
            @pl.when(p > 0)
            def _():
                pltpu.make_async_copy(*store_args(q, b)).wait()
            for a in gather_args(q, b):
                pltpu.async_copy(*a)
        for b in range(NBUF):
            q = NBUF * p + b
            for a in gather_args(q, b):
                pltpu.make_async_copy(*a).wait()
            pltpu.async_copy(*store_args(q, b))
        return carry

    lax.fori_loop(0, RING_ITERS, turn, 0)
    for t in range(NBUF * RING_ITERS, WAVES):
        b = t % NBUF
        pltpu.make_async_copy(*store_args(t, b)).wait()
        for a in gather_args(t, b):
            pltpu.async_copy(*a)
        for a in gather_args(t, b):
            pltpu.make_async_copy(*a).wait()
        pltpu.async_copy(*store_args(t, b))
    for b in range(NBUF):
        pltpu.make_async_copy(*store_args(0, b)).wait()


def _mlp_body(e_ref, w1_ref, b1_ref, w2_ref, b2_ref, w3_ref, b3_ref, o_ref):
    h = jnp.dot(e_ref[...].astype(jnp.bfloat16), w1_ref[...],
                preferred_element_type=jnp.float32).astype(jnp.bfloat16)
    h = jnp.maximum(h + b1_ref[...], jnp.bfloat16(0.0))
    h = jnp.dot(h, w2_ref[...],
                preferred_element_type=jnp.float32).astype(jnp.bfloat16)
    h = jnp.maximum(h + b2_ref[...], jnp.bfloat16(0.0))
    o = jnp.dot(h, w3_ref[...], preferred_element_type=jnp.float32)
    o_ref[...] = o + b3_ref[...]


def _mlp_table(emb, W1, b1, W2, b2, W3, b3):
    return pl.pallas_call(
        _mlp_body,
        grid=(VOCAB // TILE,),
        in_specs=[
            pl.BlockSpec((TILE, EMB_DIM), lambda i: (i, 0)),
            pl.BlockSpec((EMB_DIM, HIDDEN), lambda i: (0, 0)),
            pl.BlockSpec((1, HIDDEN), lambda i: (0, 0)),
            pl.BlockSpec((HIDDEN, HIDDEN), lambda i: (0, 0)),
            pl.BlockSpec((1, HIDDEN), lambda i: (0, 0)),
            pl.BlockSpec((HIDDEN, OUT_DIM), lambda i: (0, 0)),
            pl.BlockSpec((1, OUT_DIM), lambda i: (0, 0)),
        ],
        out_specs=pl.BlockSpec((TILE, OUT_DIM), lambda i: (i, 0)),
        out_shape=jax.ShapeDtypeStruct((VOCAB, OUT_DIM), jnp.float32),
        compiler_params=pltpu.CompilerParams(
            dimension_semantics=("parallel",)
        ),
    )(emb, W1.astype(jnp.bfloat16), b1.astype(jnp.bfloat16).reshape(1, HIDDEN),
      W2.astype(jnp.bfloat16), b2.astype(jnp.bfloat16).reshape(1, HIDDEN),
      W3.astype(jnp.bfloat16), b3.reshape(1, OUT_DIM))


def kernel(x, emb, W1, b1, W2, b2, W3, b3):
    x2 = x.reshape(IDX_ROWS, 128).astype(jnp.int32)
    tab = _mlp_table(emb, W1, b1, W2, b2, W3, b3)
    out = _sc_gather(x2, tab)
    return out.reshape(BATCH, SEQ, OUT_DIM)
